# baseline (speedup 1.0000x reference)
import functools

import jax
import jax.numpy as jnp
from jax.experimental import pallas as pl
from jax.experimental.pallas import tpu as pltpu

S = 128
K = 26
DX = 27
N_MOD = DX * DX * DX
H = 64
B = 512
LOCAL_T2 = 1.8 * 1.8
DIST_T2 = 4.5 * 4.5
DT = 1.0 / 3.0


def _moe_block(cell_ref, cur_ref, nbr_ref, idx_ref,
               wg1a_ref, wg1b_ref, bg1_ref, wg2_ref, bg2_ref,
               wla_ref, wlb_ref, bl_ref,
               wmc_ref, wmn_ref, bm_ref, wm2_ref, bm2_ref,
               wua_ref, wub_ref, bu_ref,
               wc1a_ref, wc1b_ref, bc1_ref, wc2_ref, bc2_ref,
               out_ref):
    i = pl.program_id(0)
    cur = cur_ref[...]
    idx = idx_ref[...]

    rows = jax.lax.broadcasted_iota(jnp.int32, (B, 1), 0)
    cid = (cell_ref[0] + i * B + rows) % N_MOD
    cx = cid % DX
    cy = (cid // DX) % DX
    cz = cid // (DX * DX)
    nx = idx % DX
    ny = (idx // DX) % DX
    nz = idx // (DX * DX)
    ddx = (nx - cx).astype(jnp.float32)
    ddy = (ny - cy).astype(jnp.float32)
    ddz = (nz - cz).astype(jnp.float32)
    d2 = ddx * ddx + ddy * ddy + ddz * ddz
    local_m = (d2 <= LOCAL_T2).astype(jnp.float32)
    dist_m = (d2 > DIST_T2).astype(jnp.float32)
    func_m = 1.0 - local_m - dist_m

    lc = jnp.maximum(jnp.sum(local_m, axis=1, keepdims=True), 1.0)
    dc = jnp.maximum(jnp.sum(dist_m, axis=1, keepdims=True), 1.0)
    fc = jnp.maximum(jnp.sum(func_m, axis=1, keepdims=True), 1.0)

    dot = functools.partial(jnp.dot, preferred_element_type=jnp.float32)
    cur_projb = dot(cur, wmc_ref[...]) + bm_ref[...]
    bm2 = bm2_ref[...]
    wmn = wmn_ref[...].astype(jnp.bfloat16)
    wm2 = wm2_ref[...].astype(jnp.bfloat16)

    acc = [jnp.zeros((B, S), jnp.float32) for _ in range(8)]
    for k in range(K):
        p = k & 1
        nk16 = nbr_ref[k]
        nk = nk16.astype(jnp.float32)
        acc[0 + p] = acc[0 + p] + nk
        acc[2 + p] = acc[2 + p] + local_m[:, k:k + 1] * nk
        acc[4 + p] = acc[4 + p] + dist_m[:, k:k + 1] * nk
        msg = jnp.tanh(cur_projb + dot(nk16, wmn))
        msg2 = jnp.tanh(dot(msg.astype(jnp.bfloat16), wm2) + bm2)
        acc[6 + p] = acc[6 + p] + func_m[:, k:k + 1] * msg2

    local_agg = (acc[2] + acc[3]) / lc
    dist_agg = (acc[4] + acc[5]) / dc
    func_agg = (acc[6] + acc[7]) / fc
    nbr_mean = (acc[0] + acc[1]) * (1.0 / K)

    out_local = jnp.tanh(dot(cur, wla_ref[...]) + dot(local_agg, wlb_ref[...])
                         + bl_ref[...])
    out_func = jnp.tanh(dot(cur, wua_ref[...]) + dot(func_agg, wub_ref[...])
                        + bu_ref[...])

    x = cur
    wc1a = wc1a_ref[...]
    bc1 = bc1_ref[...]
    wc2 = wc2_ref[...]
    bc2 = bc2_ref[...]
    dist_proj = dot(dist_agg, wc1b_ref[...])
    for _ in range(3):
        h = jnp.tanh(dot(x, wc1a) + dist_proj + bc1)
        x = x + DT * jnp.tanh(dot(h, wc2) + bc2)

    g = jnp.tanh(dot(cur, wg1a_ref[...]) + dot(nbr_mean, wg1b_ref[...])
                 + bg1_ref[...])
    logits = dot(g, wg2_ref[...]) + bg2_ref[...]
    m = jnp.max(logits, axis=1, keepdims=True)
    e = jnp.exp(logits - m)
    sinv = 1.0 / jnp.sum(e, axis=1, keepdims=True)
    g0 = e[:, 0:1] * sinv
    g1 = e[:, 1:2] * sinv
    g2 = e[:, 2:3] * sinv

    out_ref[...] = g0 * out_local + g1 * out_func + g2 * x


def kernel(current_state, neighbor_states, cell_idx, neighbor_indices,
           Wg1, bg1, Wg2, bg2, Wl, bl, Wm_c, Wm_n, bm, Wm2, bm2, Wu, bu,
           Wc1, bc1, Wc2, bc2):
    n = current_state.shape[0]
    grid = (n + B - 1) // B
    cell = jnp.asarray(cell_idx, jnp.int32).reshape((1,))
    idx = neighbor_indices.astype(jnp.int32)
    nbrT = jnp.swapaxes(neighbor_states, 0, 1).astype(jnp.bfloat16)

    def b2(v):
        return v.reshape(1, -1)

    def full(shape):
        return pl.BlockSpec(shape, lambda i: (0,) * len(shape))

    out = pl.pallas_call(
        _moe_block,
        grid=(grid,),
        in_specs=[
            pl.BlockSpec(memory_space=pltpu.SMEM),
            pl.BlockSpec((B, S), lambda i: (i, 0)),
            pl.BlockSpec((K, B, S), lambda i: (0, i, 0)),
            pl.BlockSpec((B, K), lambda i: (i, 0)),
            full((S, H)), full((S, H)), full((1, H)),
            full((H, 3)), full((1, 3)),
            full((S, S)), full((S, S)), full((1, S)),
            full((S, S)), full((S, S)), full((1, S)), full((S, S)), full((1, S)),
            full((S, S)), full((S, S)), full((1, S)),
            full((S, S)), full((S, S)), full((1, S)), full((S, S)), full((1, S)),
        ],
        out_specs=pl.BlockSpec((B, S), lambda i: (i, 0)),
        out_shape=jax.ShapeDtypeStruct((n, S), jnp.float32),
        compiler_params=pltpu.CompilerParams(
            dimension_semantics=("parallel",)),
    )(cell, current_state, nbrT, idx,
      Wg1[:S], Wg1[S:], b2(bg1), Wg2, b2(bg2),
      Wl[:S], Wl[S:], b2(bl),
      Wm_c, Wm_n, b2(bm), Wm2, b2(bm2),
      Wu[:S], Wu[S:], b2(bu),
      Wc1[:S], Wc1[S:], b2(bc1), Wc2, b2(bc2))
    return out


# speedup vs baseline: 1.0398x; 1.0398x over previous
import functools

import jax
import jax.numpy as jnp
from jax.experimental import pallas as pl
from jax.experimental.pallas import tpu as pltpu

S = 128
K = 26
DX = 27
N_MOD = DX * DX * DX
H = 64
B = 512
LOCAL_T2 = 1.8 * 1.8
DIST_T2 = 4.5 * 4.5
DT = 1.0 / 3.0


def _moe_block(cell_ref, cur_ref, nbr_ref, idx_ref,
               wg1a_ref, wg1b_ref, bg1_ref, wg2_ref, bg2_ref,
               wla_ref, wlb_ref, bl_ref,
               wmc_ref, wmn_ref, bm_ref, wm2_ref, bm2_ref,
               wua_ref, wub_ref, bu_ref,
               wc1a_ref, wc1b_ref, bc1_ref, wc2_ref, bc2_ref,
               out_ref):
    i = pl.program_id(0)
    cur = cur_ref[...]
    idx = idx_ref[...]

    rows = jax.lax.broadcasted_iota(jnp.int32, (B, 1), 0)
    cid = (cell_ref[0] + i * B + rows) % N_MOD
    cx = cid % DX
    cy = (cid // DX) % DX
    cz = cid // (DX * DX)
    nx = idx % DX
    ny = (idx // DX) % DX
    nz = idx // (DX * DX)
    ddx = (nx - cx).astype(jnp.float32)
    ddy = (ny - cy).astype(jnp.float32)
    ddz = (nz - cz).astype(jnp.float32)
    d2 = ddx * ddx + ddy * ddy + ddz * ddz
    local_m = (d2 <= LOCAL_T2).astype(jnp.float32)
    dist_m = (d2 > DIST_T2).astype(jnp.float32)
    func_m = 1.0 - local_m - dist_m

    lc = jnp.maximum(jnp.sum(local_m, axis=1, keepdims=True), 1.0)
    dc = jnp.maximum(jnp.sum(dist_m, axis=1, keepdims=True), 1.0)
    fc = jnp.maximum(jnp.sum(func_m, axis=1, keepdims=True), 1.0)

    dot = functools.partial(jnp.dot, preferred_element_type=jnp.float32)
    cur_projb = dot(cur, wmc_ref[...]) + bm_ref[...]
    bm2 = bm2_ref[...]
    wmn = wmn_ref[...].astype(jnp.bfloat16)
    wm2 = wm2_ref[...].astype(jnp.bfloat16)

    nbr_sum = jnp.zeros((B, S), jnp.float32)
    local_sum = jnp.zeros((B, S), jnp.float32)
    dist_sum = jnp.zeros((B, S), jnp.float32)
    func_sum = jnp.zeros((B, S), jnp.float32)
    zero = jnp.zeros((B, S), jnp.float32)
    for k in range(K):
        nk16 = nbr_ref[k]
        nk = nk16.astype(jnp.float32)
        d2b = jnp.broadcast_to(d2[:, k:k + 1], (B, S))
        nbr_sum = nbr_sum + nk
        local_sum = local_sum + jnp.where(d2b <= LOCAL_T2, nk, zero)
        dist_sum = dist_sum + jnp.where(d2b > DIST_T2, nk, zero)
        msg = jnp.tanh(cur_projb + dot(nk16, wmn))
        msg2 = jnp.tanh(dot(msg.astype(jnp.bfloat16), wm2) + bm2)
        func_sum = func_sum + jnp.where(
            (d2b > LOCAL_T2) & (d2b <= DIST_T2), msg2, zero)

    local_agg = local_sum / lc
    dist_agg = dist_sum / dc
    func_agg = func_sum / fc
    nbr_mean = nbr_sum * (1.0 / K)

    out_local = jnp.tanh(dot(cur, wla_ref[...]) + dot(local_agg, wlb_ref[...])
                         + bl_ref[...])
    out_func = jnp.tanh(dot(cur, wua_ref[...]) + dot(func_agg, wub_ref[...])
                        + bu_ref[...])

    x = cur
    wc1a = wc1a_ref[...]
    bc1 = bc1_ref[...]
    wc2 = wc2_ref[...]
    bc2 = bc2_ref[...]
    dist_proj = dot(dist_agg, wc1b_ref[...])
    for _ in range(3):
        h = jnp.tanh(dot(x, wc1a) + dist_proj + bc1)
        x = x + DT * jnp.tanh(dot(h, wc2) + bc2)

    g = jnp.tanh(dot(cur, wg1a_ref[...]) + dot(nbr_mean, wg1b_ref[...])
                 + bg1_ref[...])
    logits = dot(g, wg2_ref[...]) + bg2_ref[...]
    m = jnp.max(logits, axis=1, keepdims=True)
    e = jnp.exp(logits - m)
    sinv = 1.0 / jnp.sum(e, axis=1, keepdims=True)
    g0 = e[:, 0:1] * sinv
    g1 = e[:, 1:2] * sinv
    g2 = e[:, 2:3] * sinv

    out_ref[...] = g0 * out_local + g1 * out_func + g2 * x


def kernel(current_state, neighbor_states, cell_idx, neighbor_indices,
           Wg1, bg1, Wg2, bg2, Wl, bl, Wm_c, Wm_n, bm, Wm2, bm2, Wu, bu,
           Wc1, bc1, Wc2, bc2):
    n = current_state.shape[0]
    grid = (n + B - 1) // B
    cell = jnp.asarray(cell_idx, jnp.int32).reshape((1,))
    idx = neighbor_indices.astype(jnp.int32)
    nbrT = jnp.swapaxes(neighbor_states, 0, 1).astype(jnp.bfloat16)

    def b2(v):
        return v.reshape(1, -1)

    def full(shape):
        return pl.BlockSpec(shape, lambda i: (0,) * len(shape))

    out = pl.pallas_call(
        _moe_block,
        grid=(grid,),
        in_specs=[
            pl.BlockSpec(memory_space=pltpu.SMEM),
            pl.BlockSpec((B, S), lambda i: (i, 0)),
            pl.BlockSpec((K, B, S), lambda i: (0, i, 0)),
            pl.BlockSpec((B, K), lambda i: (i, 0)),
            full((S, H)), full((S, H)), full((1, H)),
            full((H, 3)), full((1, 3)),
            full((S, S)), full((S, S)), full((1, S)),
            full((S, S)), full((S, S)), full((1, S)), full((S, S)), full((1, S)),
            full((S, S)), full((S, S)), full((1, S)),
            full((S, S)), full((S, S)), full((1, S)), full((S, S)), full((1, S)),
        ],
        out_specs=pl.BlockSpec((B, S), lambda i: (i, 0)),
        out_shape=jax.ShapeDtypeStruct((n, S), jnp.float32),
        compiler_params=pltpu.CompilerParams(
            dimension_semantics=("parallel",)),
    )(cell, current_state, nbrT, idx,
      Wg1[:S], Wg1[S:], b2(bg1), Wg2, b2(bg2),
      Wl[:S], Wl[S:], b2(bl),
      Wm_c, Wm_n, b2(bm), Wm2, b2(bm2),
      Wu[:S], Wu[S:], b2(bu),
      Wc1[:S], Wc1[S:], b2(bc1), Wc2, b2(bc2))
    return out


# speedup vs baseline: 1.1346x; 1.0913x over previous
import functools

import jax
import jax.numpy as jnp
from jax.experimental import pallas as pl
from jax.experimental.pallas import tpu as pltpu

S = 128
K = 26
DX = 27
N_MOD = DX * DX * DX
H = 64
B = 512
LOCAL_T2 = 1.8 * 1.8
DIST_T2 = 4.5 * 4.5
DT = 1.0 / 3.0


def _moe_block(cell_ref, cur_ref, nbr_ref, idx_ref,
               wg1a_ref, wg1b_ref, bg1_ref, wg2_ref, bg2_ref,
               wla_ref, wlb_ref, bl_ref,
               wmc_ref, wmn_ref, bm_ref, wm2_ref, bm2_ref,
               wua_ref, wub_ref, bu_ref,
               wc1a_ref, wc1b_ref, bc1_ref, wc2_ref, bc2_ref,
               out_ref):
    i = pl.program_id(0)
    cur = cur_ref[...]
    idx = idx_ref[...]

    rows = jax.lax.broadcasted_iota(jnp.int32, (B, 1), 0)
    cid = (cell_ref[0] + i * B + rows) % N_MOD
    cx = cid % DX
    cy = (cid // DX) % DX
    cz = cid // (DX * DX)
    nx = idx % DX
    ny = (idx // DX) % DX
    nz = idx // (DX * DX)
    ddx = (nx - cx).astype(jnp.float32)
    ddy = (ny - cy).astype(jnp.float32)
    ddz = (nz - cz).astype(jnp.float32)
    d2 = ddx * ddx + ddy * ddy + ddz * ddz
    local_m = (d2 <= LOCAL_T2).astype(jnp.float32)
    dist_m = (d2 > DIST_T2).astype(jnp.float32)
    func_m = 1.0 - local_m - dist_m

    lc = jnp.maximum(jnp.sum(local_m, axis=1, keepdims=True), 1.0)
    dc = jnp.maximum(jnp.sum(dist_m, axis=1, keepdims=True), 1.0)
    fc = jnp.maximum(jnp.sum(func_m, axis=1, keepdims=True), 1.0)

    dot = functools.partial(jnp.dot, preferred_element_type=jnp.float32)
    cur_projb = dot(cur, wmc_ref[...]) + bm_ref[...]
    bm2 = bm2_ref[...]
    wmn = wmn_ref[...].astype(jnp.bfloat16)
    wm2 = wm2_ref[...].astype(jnp.bfloat16)

    bf = jnp.bfloat16
    nbr_sum = jnp.zeros((B, S), bf)
    local_sum = jnp.zeros((B, S), bf)
    dist_sum = jnp.zeros((B, S), bf)
    func_sum = jnp.zeros((B, S), bf)
    zero = jnp.zeros((B, S), bf)
    d2_16 = d2.astype(bf)
    for k in range(K):
        nk16 = nbr_ref[k]
        d2b = jnp.broadcast_to(d2_16[:, k:k + 1], (B, S))
        nbr_sum = nbr_sum + nk16
        local_sum = local_sum + jnp.where(d2b <= bf(LOCAL_T2), nk16, zero)
        dist_sum = dist_sum + jnp.where(d2b > bf(DIST_T2), nk16, zero)
        msg = jnp.tanh(cur_projb + dot(nk16, wmn))
        msg2 = jnp.tanh(dot(msg.astype(bf), wm2) + bm2).astype(bf)
        func_sum = func_sum + jnp.where(
            (d2b > bf(LOCAL_T2)) & (d2b <= bf(DIST_T2)), msg2, zero)

    local_agg = local_sum.astype(jnp.float32) / lc
    dist_agg = dist_sum.astype(jnp.float32) / dc
    func_agg = func_sum.astype(jnp.float32) / fc
    nbr_mean = nbr_sum.astype(jnp.float32) * (1.0 / K)

    out_local = jnp.tanh(dot(cur, wla_ref[...]) + dot(local_agg, wlb_ref[...])
                         + bl_ref[...])
    out_func = jnp.tanh(dot(cur, wua_ref[...]) + dot(func_agg, wub_ref[...])
                        + bu_ref[...])

    x = cur
    wc1a = wc1a_ref[...]
    bc1 = bc1_ref[...]
    wc2 = wc2_ref[...]
    bc2 = bc2_ref[...]
    dist_proj = dot(dist_agg, wc1b_ref[...])
    for _ in range(3):
        h = jnp.tanh(dot(x, wc1a) + dist_proj + bc1)
        x = x + DT * jnp.tanh(dot(h, wc2) + bc2)

    g = jnp.tanh(dot(cur, wg1a_ref[...]) + dot(nbr_mean, wg1b_ref[...])
                 + bg1_ref[...])
    logits = dot(g, wg2_ref[...]) + bg2_ref[...]
    m = jnp.max(logits, axis=1, keepdims=True)
    e = jnp.exp(logits - m)
    sinv = 1.0 / jnp.sum(e, axis=1, keepdims=True)
    g0 = e[:, 0:1] * sinv
    g1 = e[:, 1:2] * sinv
    g2 = e[:, 2:3] * sinv

    out_ref[...] = g0 * out_local + g1 * out_func + g2 * x


def kernel(current_state, neighbor_states, cell_idx, neighbor_indices,
           Wg1, bg1, Wg2, bg2, Wl, bl, Wm_c, Wm_n, bm, Wm2, bm2, Wu, bu,
           Wc1, bc1, Wc2, bc2):
    n = current_state.shape[0]
    grid = (n + B - 1) // B
    cell = jnp.asarray(cell_idx, jnp.int32).reshape((1,))
    idx = neighbor_indices.astype(jnp.int32)
    nbrT = jnp.swapaxes(neighbor_states, 0, 1).astype(jnp.bfloat16)

    def b2(v):
        return v.reshape(1, -1)

    def full(shape):
        return pl.BlockSpec(shape, lambda i: (0,) * len(shape))

    out = pl.pallas_call(
        _moe_block,
        grid=(grid,),
        in_specs=[
            pl.BlockSpec(memory_space=pltpu.SMEM),
            pl.BlockSpec((B, S), lambda i: (i, 0)),
            pl.BlockSpec((K, B, S), lambda i: (0, i, 0)),
            pl.BlockSpec((B, K), lambda i: (i, 0)),
            full((S, H)), full((S, H)), full((1, H)),
            full((H, 3)), full((1, 3)),
            full((S, S)), full((S, S)), full((1, S)),
            full((S, S)), full((S, S)), full((1, S)), full((S, S)), full((1, S)),
            full((S, S)), full((S, S)), full((1, S)),
            full((S, S)), full((S, S)), full((1, S)), full((S, S)), full((1, S)),
        ],
        out_specs=pl.BlockSpec((B, S), lambda i: (i, 0)),
        out_shape=jax.ShapeDtypeStruct((n, S), jnp.float32),
        compiler_params=pltpu.CompilerParams(
            dimension_semantics=("parallel",)),
    )(cell, current_state, nbrT, idx,
      Wg1[:S], Wg1[S:], b2(bg1), Wg2, b2(bg2),
      Wl[:S], Wl[S:], b2(bl),
      Wm_c, Wm_n, b2(bm), Wm2, b2(bm2),
      Wu[:S], Wu[S:], b2(bu),
      Wc1[:S], Wc1[S:], b2(bc1), Wc2, b2(bc2))
    return out
